# baseline (speedup 1.0000x reference)
import functools

import jax
import jax.numpy as jnp
from jax import lax
from jax.experimental import pallas as pl
from jax.experimental.pallas import tpu as pltpu
from jax.experimental.pallas import tpu_sc as plsc

VOCAB = 1000000
D = 64
DP = 128
SEQ = 200
BATCH = 4096
N = BATCH * SEQ

NC, NS = 2, 16
NW = NC * NS
BBLK = BATCH // NW
TS = 2
NSC = SEQ // TS


def _make_k():
    mesh = plsc.VectorSubcoreMesh(core_axis_name="c", subcore_axis_name="s")

    @functools.partial(
        pl.kernel,
        mesh=mesh,
        out_type=jax.ShapeDtypeStruct((SEQ, 8, NW, 1024), jnp.float32),
        compiler_params=pltpu.CompilerParams(
            use_tc_tiling_on_sc=False, needs_layout_passes=False
        ),
        scratch_types=[
            pltpu.VMEM((2, TS, 128), jnp.int32),
            pltpu.VMEM((2, TS * 128, D), jnp.float32),
            pltpu.VMEM((2, TS * 8, 1024), jnp.float32),
            pltpu.VMEM((SEQ, D), jnp.float32),
            pltpu.SemaphoreType.DMA,
            pltpu.SemaphoreType.DMA,
            pltpu.SemaphoreType.DMA,
            pltpu.SemaphoreType.DMA,
        ],
    )
    def k(tok2_hbm, xt_hbm, pos_hbm, out_hbm, idx_v, rows_v, tv, pos_v,
          sem0, sem1, semw0, semw1):
        wid = lax.axis_index("s") * NC + lax.axis_index("c")
        b0 = wid * BBLK
        i16 = lax.iota(jnp.int32, 16)
        sems = [sem0, sem1]
        semws = [semw0, semw1]

        pltpu.sync_copy(pos_hbm, pos_v)

        def fire(g, p):
            s0 = g * TS
            pltpu.sync_copy(
                xt_hbm.at[pl.ds(s0, TS), pl.ds(b0, 128)], idx_v.at[p]
            )

            def dbl_body(t, c):
                for q in range(8):
                    idx_v[p, t, pl.ds(16 * q, 16)] = (
                        idx_v[p, t, pl.ds(16 * q, 16)] * 2
                    )
                return c

            lax.fori_loop(0, TS, dbl_body, 0)
            for si in range(TS):
                pltpu.async_copy(
                    tok2_hbm.at[idx_v.at[p, si]],
                    rows_v.at[p, pl.ds(si * 128, 128)],
                    sems[p],
                )

        def drain(p):
            for si in range(TS):
                pltpu.make_async_copy(
                    tok2_hbm.at[idx_v.at[p, si]],
                    rows_v.at[p, pl.ds(si * 128, 128)],
                    sems[p],
                ).wait()

        def transpose(g, p):
            s0 = g * TS
            for si in range(TS):
                pvs = [pos_v[s0 + si, pl.ds(16 * j, 16)] for j in range(4)]
                rowj = [si * 8 + (16 * j + i16) // 8 for j in range(4)]
                colj = lax.rem(i16, 8) * 128

                def r_body(r, c):
                    col = colj + r
                    for j in range(4):
                        vals = (
                            rows_v[p, si * 128 + r, pl.ds(16 * j, 16)]
                            + pvs[j]
                        )
                        plsc.store_scatter(tv.at[p], [rowj[j], col], vals)
                    return c

                lax.fori_loop(0, 128, r_body, 0)

        def fire_writes(g, p):
            s0 = g * TS
            for si in range(TS):
                pltpu.async_copy(
                    tv.at[p, pl.ds(si * 8, 8), :],
                    out_hbm.at[s0 + si, :, wid],
                    semws[p],
                )

        def drain_writes(g, p):
            s0 = g * TS
            for si in range(TS):
                pltpu.make_async_copy(
                    tv.at[p, pl.ds(si * 8, 8), :],
                    out_hbm.at[s0 + si, :, wid],
                    semws[p],
                ).wait()

        fire(0, 0)

        def pair_body(h, carry):
            g0 = 2 * h
            drain(0)
            fire(g0 + 1, 1)
            transpose(g0, 0)

            @pl.when(h > 0)
            def _():
                drain_writes(g0 - 1, 1)

            fire_writes(g0, 0)
            drain(1)

            @pl.when(g0 + 2 < NSC)
            def _():
                fire(g0 + 2, 0)

            transpose(g0 + 1, 1)
            drain_writes(g0, 0)
            fire_writes(g0 + 1, 1)
            return carry

        lax.fori_loop(0, NSC // 2, pair_body, 0)
        drain_writes(NSC - 1, 1)

    return k


_k = _make_k()


@jax.jit
def kernel(x, token_table, pos_table):
    tok128 = jnp.pad(token_table, ((0, 0), (0, DP - D)))
    tok2 = tok128.reshape(2 * VOCAB, D)
    xt = x.T
    out4 = _k(tok2, xt, pos_table)
    out5 = out4.reshape(SEQ, 8, NW, 8, 128)
    out = out5.transpose(2, 4, 0, 1, 3).reshape(BATCH, SEQ, D)
    return out


# speedup vs baseline: 1.7803x; 1.7803x over previous
import functools

import jax
import jax.numpy as jnp
from jax import lax
from jax.experimental import pallas as pl
from jax.experimental.pallas import tpu as pltpu
from jax.experimental.pallas import tpu_sc as plsc

VOCAB = 1000000
D = 64
DP = 128
SEQ = 200
BATCH = 4096
N = BATCH * SEQ

NC, NS = 2, 16
NW = NC * NS
PER_W = N // NW
SEQ_PER_CHUNK = 4
CHUNK = SEQ * SEQ_PER_CHUNK
NCHUNK = PER_W // CHUNK
GATHER_W = 100
NGATHER = CHUNK // GATHER_W


def _make_kernel():
    mesh = plsc.VectorSubcoreMesh(core_axis_name="c", subcore_axis_name="s")

    @functools.partial(
        pl.kernel,
        mesh=mesh,
        out_type=jax.ShapeDtypeStruct((N, DP), jnp.float32),
        compiler_params=pltpu.CompilerParams(use_tc_tiling_on_sc=False),
        scratch_types=[
            pltpu.VMEM((NGATHER, GATHER_W), jnp.int32),
            pltpu.VMEM((CHUNK, D), jnp.float32),
            pltpu.VMEM((SEQ, D), jnp.float32),
            pltpu.SemaphoreType.DMA,
        ],
    )
    def emb_kernel(tok_hbm, xf_hbm, pos_hbm, out_hbm, idx_v, rows_v, pos_v, sem):
        wid = lax.axis_index("s") * NC + lax.axis_index("c")
        base = wid * PER_W

        pltpu.sync_copy(pos_hbm, pos_v)

        def chunk_body(g, carry):
            start = base + g * CHUNK
            row0 = pl.multiple_of(start // GATHER_W, NGATHER)
            pltpu.sync_copy(xf_hbm.at[pl.ds(row0, NGATHER)], idx_v)
            for j in range(NGATHER):
                pltpu.async_copy(
                    tok_hbm.at[idx_v.at[j]],
                    rows_v.at[pl.ds(j * GATHER_W, GATHER_W)],
                    sem,
                )
            for j in range(NGATHER):
                pltpu.make_async_copy(
                    tok_hbm.at[idx_v.at[j]],
                    rows_v.at[pl.ds(j * GATHER_W, GATHER_W)],
                    sem,
                ).wait()

            def pos_body(s, c):
                for j in range(D // 16):
                    pv = pos_v[s, pl.ds(16 * j, 16)]
                    for kk in range(SEQ_PER_CHUNK):
                        plsc.addupdate(
                            rows_v.at[kk * SEQ + s, pl.ds(16 * j, 16)], pv
                        )
                return c

            lax.fori_loop(0, SEQ, pos_body, 0)

            pltpu.sync_copy(
                rows_v,
                out_hbm.at[pl.ds(start, CHUNK), pl.ds(0, D)],
            )
            return carry

        lax.fori_loop(0, NCHUNK, chunk_body, 0)

    return emb_kernel


_emb_kernel = _make_kernel()


@jax.jit
def kernel(x, token_table, pos_table):
    tok128 = jnp.pad(token_table, ((0, 0), (0, DP - D)))
    tok2 = tok128.reshape(2 * VOCAB, D)
    xf = (x.reshape(N // GATHER_W, GATHER_W) * 2).astype(jnp.int32)
    out = _emb_kernel(tok2, xf, pos_table)
    return out.reshape(BATCH, SEQ, DP)[:, :, :D]
